# baseline (speedup 1.0000x reference)
import functools

import jax
import jax.numpy as jnp
from jax import lax
from jax.experimental import pallas as pl
from jax.experimental.pallas import tpu as pltpu
from jax.experimental.pallas import tpu_sc as plsc

_NC = 2
_NS = 16
_NW = _NC * _NS
_CH = 128
_NBUF = 5


@functools.cache
def _build(V, D, n_chunks):
    per_w = n_chunks * _CH
    mesh = plsc.VectorSubcoreMesh(core_axis_name="c", subcore_axis_name="s")

    @functools.partial(
        pl.kernel,
        out_type=jax.ShapeDtypeStruct((_NW * per_w, D), jnp.float32),
        mesh=mesh,
        scratch_types=[
            pltpu.VMEM((n_chunks + _NBUF, _CH), jnp.int32),
            [pltpu.VMEM((_CH, D), jnp.float32)] * _NBUF,
            [pltpu.SemaphoreType.DMA] * _NBUF,
            [pltpu.SemaphoreType.DMA] * _NBUF,
        ],
    )
    def k(idx_hbm, table_hbm, out_hbm, idx_v, bufs, gsems, wsems):
        wid = lax.axis_index("s") * _NC + lax.axis_index("c")
        base = wid * per_w

        def wait_gather(b):
            pltpu.make_async_copy(
                table_hbm.at[pl.ds(0, _CH)], bufs[b], gsems[b]).wait()

        def wait_write(b):
            pltpu.make_async_copy(
                bufs[b], out_hbm.at[pl.ds(base, _CH)], wsems[b]).wait()

        pltpu.sync_copy(idx_hbm.at[wid], idx_v)
        for b in range(_NBUF):
            pltpu.async_copy(table_hbm.at[idx_v.at[b]], bufs[b], gsems[b])

        @pl.loop(0, n_chunks, step=_NBUF)
        def _(i):
            for b in range(_NBUF):
                wait_gather(b)
                pltpu.async_copy(
                    bufs[b], out_hbm.at[pl.ds(base + (i + b) * _CH, _CH)],
                    wsems[b])
            for b in range(_NBUF):
                wait_write(b)
                pltpu.async_copy(
                    table_hbm.at[idx_v.at[i + b + _NBUF]], bufs[b], gsems[b])

        for b in range(_NBUF):
            wait_gather(b)

    return k


def kernel(seqs, table):
    B, H = seqs.shape
    V, D = table.shape
    flat = seqs.reshape(-1).astype(jnp.int32)
    n = flat.shape[0]
    assert n % (_NW * _CH) == 0 and (n // (_NW * _CH)) % _NBUF == 0
    n_chunks = n // (_NW * _CH)
    idx = flat.reshape(_NW, n_chunks, _CH)
    idx = jnp.pad(idx, ((0, 0), (0, _NBUF), (0, 0)))
    out = _build(V, D, n_chunks)(idx, table)
    return out.reshape(B, H, D)


# speedup vs baseline: 2.3859x; 2.3859x over previous
import functools

import jax
import jax.numpy as jnp
from jax import lax
from jax.experimental import pallas as pl
from jax.experimental.pallas import tpu as pltpu
from jax.experimental.pallas import tpu_sc as plsc

_NC = 2
_NS = 16
_NW = _NC * _NS
_CH = 128


@functools.cache
def _build(V, D, n_chunks):
    per_w = n_chunks * _CH
    mesh = plsc.VectorSubcoreMesh(core_axis_name="c", subcore_axis_name="s")

    @functools.partial(
        pl.kernel,
        out_type=jax.ShapeDtypeStruct((_NW * per_w, D), jnp.float32),
        mesh=mesh,
        scratch_types=[
            pltpu.VMEM((n_chunks + 2, _CH), jnp.int32),
            pltpu.VMEM((_CH, D), jnp.float32),
            pltpu.VMEM((_CH, D), jnp.float32),
            pltpu.SemaphoreType.DMA,
            pltpu.SemaphoreType.DMA,
        ],
    )
    def k(idx_hbm, table_hbm, out_hbm, idx_v, rows0, rows1, sem0, sem1):
        wid = lax.axis_index("s") * _NC + lax.axis_index("c")
        base = wid * per_w

        def wait_gather(rows, sem):
            pltpu.make_async_copy(table_hbm.at[pl.ds(0, _CH)], rows, sem).wait()

        pltpu.sync_copy(idx_hbm.at[wid], idx_v)
        pltpu.async_copy(table_hbm.at[idx_v.at[0]], rows0, sem0)

        @pl.loop(0, n_chunks, step=2)
        def _(i):
            pltpu.async_copy(table_hbm.at[idx_v.at[i + 1]], rows1, sem1)
            wait_gather(rows0, sem0)
            pltpu.sync_copy(rows0, out_hbm.at[pl.ds(base + i * _CH, _CH)])
            pltpu.async_copy(table_hbm.at[idx_v.at[i + 2]], rows0, sem0)
            wait_gather(rows1, sem1)
            pltpu.sync_copy(rows1, out_hbm.at[pl.ds(base + (i + 1) * _CH, _CH)])

        wait_gather(rows0, sem0)

    return k


def kernel(seqs, table):
    B, H = seqs.shape
    V, D = table.shape
    flat = seqs.reshape(-1).astype(jnp.int32)
    n = flat.shape[0]
    assert n % (_NW * _CH) == 0 and n // (_NW * _CH) % 2 == 0
    n_chunks = n // (_NW * _CH)
    idx = flat.reshape(_NW, n_chunks, _CH)
    idx = jnp.pad(idx, ((0, 0), (0, 2), (0, 0)))
    out = _build(V, D, n_chunks)(idx, table)
    return out.reshape(B, H, D)


# speedup vs baseline: 3.6676x; 1.5372x over previous
import functools

import jax
import jax.numpy as jnp
from jax import lax
from jax.experimental import pallas as pl
from jax.experimental.pallas import tpu as pltpu
from jax.experimental.pallas import tpu_sc as plsc

_NC = 2
_NS = 16
_NW = _NC * _NS
_CH = 128


@functools.cache
def _build(V, D, n_chunks):
    per_w = n_chunks * _CH
    mesh = plsc.VectorSubcoreMesh(core_axis_name="c", subcore_axis_name="s")

    @functools.partial(
        pl.kernel,
        out_type=jax.ShapeDtypeStruct((_NW * per_w, D), jnp.float32),
        mesh=mesh,
        scratch_types=[
            pltpu.VMEM((n_chunks, _CH), jnp.int32),
            pltpu.VMEM((_CH, D), jnp.float32),
            pltpu.VMEM((_CH, D), jnp.float32),
            pltpu.SemaphoreType.DMA,
            pltpu.SemaphoreType.DMA,
        ],
    )
    def k(idx_hbm, table_hbm, out_hbm, idx_v, rows0, rows1, sem0, sem1):
        wid = lax.axis_index("s") * _NC + lax.axis_index("c")
        base = wid * per_w

        def wait_gather(rows, sem):
            pltpu.make_async_copy(table_hbm.at[pl.ds(0, _CH)], rows, sem).wait()

        pltpu.sync_copy(idx_hbm.at[wid], idx_v)
        pltpu.async_copy(table_hbm.at[idx_v.at[0]], rows0, sem0)

        @pl.loop(0, n_chunks - 2, step=2)
        def _(i):
            pltpu.async_copy(table_hbm.at[idx_v.at[i + 1]], rows1, sem1)
            wait_gather(rows0, sem0)
            pltpu.sync_copy(rows0, out_hbm.at[pl.ds(base + i * _CH, _CH)])
            pltpu.async_copy(table_hbm.at[idx_v.at[i + 2]], rows0, sem0)
            wait_gather(rows1, sem1)
            pltpu.sync_copy(rows1, out_hbm.at[pl.ds(base + (i + 1) * _CH, _CH)])

        pltpu.async_copy(table_hbm.at[idx_v.at[n_chunks - 1]], rows1, sem1)
        wait_gather(rows0, sem0)
        pltpu.sync_copy(
            rows0, out_hbm.at[pl.ds(base + (n_chunks - 2) * _CH, _CH)])
        wait_gather(rows1, sem1)
        pltpu.sync_copy(
            rows1, out_hbm.at[pl.ds(base + (n_chunks - 1) * _CH, _CH)])

    return k


def kernel(seqs, table):
    B, H = seqs.shape
    V, D = table.shape
    flat = seqs.reshape(-1).astype(jnp.int32)
    n = flat.shape[0]
    assert n % (_NW * _CH) == 0 and n // (_NW * _CH) % 2 == 0
    n_chunks = n // (_NW * _CH)
    idx = flat.reshape(_NW, n_chunks, _CH)
    out = _build(V, D, n_chunks)(idx, table)
    return out.reshape(B, H, D)


# speedup vs baseline: 3.6892x; 1.0059x over previous
import functools

import jax
import jax.numpy as jnp
from jax import lax
from jax.experimental import pallas as pl
from jax.experimental.pallas import tpu as pltpu
from jax.experimental.pallas import tpu_sc as plsc

_NC = 2
_NS = 16
_NW = _NC * _NS
_CH = 128
_NB = 5
_LA = 2


@functools.cache
def _build(V, D, n_chunks):
    per_w = n_chunks * _CH
    n_loop = ((n_chunks - _NB - _LA) // _NB) * _NB
    n_epi = n_chunks - _NB - n_loop
    mesh = plsc.VectorSubcoreMesh(core_axis_name="c", subcore_axis_name="s")

    @functools.partial(
        pl.kernel,
        out_type=jax.ShapeDtypeStruct((_NW * per_w, D), jnp.float32),
        mesh=mesh,
        scratch_types=[
            pltpu.VMEM((n_chunks, _CH), jnp.int32),
            [pltpu.VMEM((_CH, D), jnp.float32) for _ in range(_NB)],
            [pltpu.SemaphoreType.DMA for _ in range(_NB)],
            [pltpu.SemaphoreType.DMA for _ in range(_NB)],
        ],
    )
    def k(idx_hbm, table_hbm, out_hbm, idx_v, bufs, gsems, wsems):
        wid = lax.axis_index("s") * _NC + lax.axis_index("c")
        base = wid * per_w

        def gather(c, b):
            pltpu.async_copy(table_hbm.at[idx_v.at[c]], bufs[b], gsems[b])

        def wait_gather(b):
            pltpu.make_async_copy(
                table_hbm.at[pl.ds(0, _CH)], bufs[b], gsems[b]).wait()

        def write(c, b):
            pltpu.async_copy(
                bufs[b], out_hbm.at[pl.ds(base + c * _CH, _CH)], wsems[b])

        def wait_write(b):
            pltpu.make_async_copy(
                bufs[b], out_hbm.at[pl.ds(base, _CH)], wsems[b]).wait()

        def step(c, b, refill_c, need_wwait):
            wait_gather(b)
            write(c, b)
            if refill_c is not None:
                b2 = (b + _LA) % _NB
                if need_wwait:
                    wait_write(b2)
                gather(refill_c, b2)

        pltpu.sync_copy(idx_hbm.at[wid], idx_v)
        for c in range(_LA):
            gather(c, c % _NB)
        for c in range(_NB):
            step(c, c % _NB, c + _LA, c >= _NB - _LA)

        @pl.loop(_NB, _NB + n_loop, step=_NB)
        def _(i):
            for b in range(_NB):
                step(i + b, b, i + b + _LA, True)

        for e in range(n_epi):
            c = _NB + n_loop + e
            rc = c + _LA
            step(c, c % _NB, rc if rc < n_chunks else None, True)
        for c in range(n_chunks - _NB, n_chunks):
            wait_write(c % _NB)

    return k


def kernel(seqs, table):
    B, H = seqs.shape
    V, D = table.shape
    flat = seqs.reshape(-1).astype(jnp.int32)
    n = flat.shape[0]
    assert n % (_NW * _CH) == 0
    n_chunks = n // (_NW * _CH)
    assert n_chunks >= _NB + _LA
    idx = flat.reshape(_NW, n_chunks, _CH)
    out = _build(V, D, n_chunks)(idx, table)
    return out.reshape(B, H, D)
